# Initial kernel scaffold; baseline (speedup 1.0000x reference)
import jax
import jax.numpy as jnp
from jax.experimental import pallas as pl


def kernel(x, weights, node_order):
    raise NotImplementedError("write your pallas kernel here")


# speedup vs baseline: 1.7195x; 1.7195x over previous
import jax
import jax.numpy as jnp
from jax import lax
from jax.experimental import pallas as pl
from jax.experimental.pallas import tpu as pltpu
from jax.experimental.pallas import tpu_sc as plsc

L = 16
NC = 2
NS = 16
NW = NC * NS
CHUNK = 256
NBUF = 2
SIZE_OUT = 128


def _sc_body(xf_hbm, w_hbm, no_hbm, out_hbm, w_v, no_v, xbuf, obuf,
             insem, outsem):
    size_in = w_hbm.shape[0]
    nvec_in = size_in // L
    batch = xf_hbm.shape[0] // size_in
    rows_per_w = batch // NW
    nch = rows_per_w // CHUNK

    wid = lax.axis_index("s") * NC + lax.axis_index("c")
    base = wid * rows_per_w

    pltpu.sync_copy(w_hbm, w_v)
    pltpu.sync_copy(no_hbm, no_v)

    zeros = jnp.zeros((L,), jnp.float32)

    @pl.loop(0, NBUF * CHUNK * SIZE_OUT // L, unroll=8)
    def _(i):
        obuf[pl.ds(i * L, L)] = zeros

    wvecs = [w_v[pl.ds(j * L, L)] for j in range(nvec_in)]
    cvecs = [no_v[pl.ds(j * L, L)] for j in range(nvec_in)]

    in_cp = [
        pltpu.make_async_copy(
            xf_hbm.at[pl.ds((base + ch * CHUNK) * size_in, CHUNK * size_in)],
            xbuf.at[pl.ds((ch % NBUF) * CHUNK * size_in, CHUNK * size_in)],
            insem.at[ch % NBUF])
        for ch in range(nch)
    ]
    out_cp = [
        pltpu.make_async_copy(
            obuf.at[pl.ds((ch % NBUF) * CHUNK * SIZE_OUT, CHUNK * SIZE_OUT)],
            out_hbm.at[pl.ds((base + ch * CHUNK) * SIZE_OUT,
                             CHUNK * SIZE_OUT)],
            outsem.at[ch % NBUF])
        for ch in range(nch)
    ]

    in_cp[0].start()
    for ch in range(nch):
        b = ch % NBUF
        if ch + 1 < nch:
            in_cp[ch + 1].start()
        in_cp[ch].wait()
        if ch >= NBUF:
            out_cp[ch - NBUF].wait()

        xoff = b * CHUNK * size_in
        ooff = b * CHUNK * SIZE_OUT

        @pl.loop(0, CHUNK)
        def _(r):
            rbase = jnp.full((L,), ooff + r * SIZE_OUT, jnp.int32)
            for j in range(nvec_in):
                v = xbuf[pl.ds(xoff + r * size_in + j * L, L)] * wvecs[j]
                plsc.store_scatter(obuf, [rbase + cvecs[j]], v)

        out_cp[ch].start()
    for ch in range(max(0, nch - NBUF), nch):
        out_cp[ch].wait()


@jax.jit
def kernel(x, weights, node_order):
    B, size_in = x.shape
    mesh = plsc.VectorSubcoreMesh(core_axis_name="c", subcore_axis_name="s")
    f = pl.kernel(
        _sc_body,
        out_type=jax.ShapeDtypeStruct((B * SIZE_OUT,), x.dtype),
        mesh=mesh,
        compiler_params=pltpu.CompilerParams(needs_layout_passes=False),
        scratch_types=[
            pltpu.VMEM((size_in,), jnp.float32),
            pltpu.VMEM((size_in,), jnp.int32),
            pltpu.VMEM((NBUF * CHUNK * size_in,), jnp.float32),
            pltpu.VMEM((NBUF * CHUNK * SIZE_OUT,), jnp.float32),
            pltpu.SemaphoreType.DMA((NBUF,)),
            pltpu.SemaphoreType.DMA((NBUF,)),
        ],
    )
    out = f(x.reshape(B * size_in), weights, node_order)
    return out.reshape(B, SIZE_OUT)


# speedup vs baseline: 2.6775x; 1.5571x over previous
import jax
import jax.numpy as jnp
from jax import lax
from jax.experimental import pallas as pl
from jax.experimental.pallas import tpu as pltpu
from jax.experimental.pallas import tpu_sc as plsc

L = 16
NC = 2
NS = 16
NW = NC * NS
CHUNK = 128
NBUF = 2
SIZE_OUT = 128


def _sc_body(x_hbm, w_hbm, no_hbm, out_hbm, w_v, no_v, xbufs, obufs,
             insem, outsem):
    size_in = x_hbm.shape[1]
    nvec_in = size_in // L
    rows_per_w = x_hbm.shape[0] // NW
    nch = rows_per_w // CHUNK

    wid = lax.axis_index("s") * NC + lax.axis_index("c")
    base = wid * rows_per_w

    pltpu.sync_copy(w_hbm, w_v)
    pltpu.sync_copy(no_hbm, no_v)

    zeros = jnp.zeros((L,), jnp.float32)
    for b in range(NBUF):
        @plsc.parallel_loop(0, CHUNK, unroll=4)
        def _(r, b=b):
            for k in range(SIZE_OUT // L):
                obufs[b][r, pl.ds(k * L, L)] = zeros

    wvecs = [w_v[pl.ds(j * L, L)] for j in range(nvec_in)]
    cvecs = [no_v[pl.ds(j * L, L)] for j in range(nvec_in)]

    in_cp = [
        pltpu.make_async_copy(
            x_hbm.at[pl.ds(base + ch * CHUNK, CHUNK)], xbufs[ch % NBUF],
            insem.at[ch % NBUF])
        for ch in range(nch)
    ]
    out_cp = [
        pltpu.make_async_copy(
            obufs[ch % NBUF], out_hbm.at[pl.ds(base + ch * CHUNK, CHUNK)],
            outsem.at[ch % NBUF])
        for ch in range(nch)
    ]

    in_cp[0].start()
    for ch in range(nch):
        b = ch % NBUF
        if ch + 1 < nch:
            in_cp[ch + 1].start()
        in_cp[ch].wait()
        if ch >= NBUF:
            out_cp[ch - NBUF].wait()

        @plsc.parallel_loop(0, CHUNK, unroll=4)
        def _(r, b=b):
            ridx = jnp.full((L,), r, jnp.int32)
            for j in range(nvec_in):
                v = xbufs[b][r, pl.ds(j * L, L)] * wvecs[j]
                plsc.store_scatter(obufs[b], [ridx, cvecs[j]], v)

        out_cp[ch].start()
    for ch in range(max(0, nch - NBUF), nch):
        out_cp[ch].wait()


def _body(x_hbm, w_hbm, no_hbm, out_hbm, w_v, no_v, xbuf0, xbuf1,
          obuf0, obuf1, insem, outsem):
    _sc_body(x_hbm, w_hbm, no_hbm, out_hbm, w_v, no_v,
             (xbuf0, xbuf1), (obuf0, obuf1), insem, outsem)


@jax.jit
def kernel(x, weights, node_order):
    B = x.shape[0]
    mesh = plsc.VectorSubcoreMesh(core_axis_name="c", subcore_axis_name="s")
    f = pl.kernel(
        _body,
        out_type=jax.ShapeDtypeStruct((B, SIZE_OUT), x.dtype),
        mesh=mesh,
        compiler_params=pltpu.CompilerParams(needs_layout_passes=False),
        scratch_types=[
            pltpu.VMEM((x.shape[1],), jnp.float32),
            pltpu.VMEM((x.shape[1],), jnp.int32),
            pltpu.VMEM((CHUNK, x.shape[1]), jnp.float32),
            pltpu.VMEM((CHUNK, x.shape[1]), jnp.float32),
            pltpu.VMEM((CHUNK, SIZE_OUT), jnp.float32),
            pltpu.VMEM((CHUNK, SIZE_OUT), jnp.float32),
            pltpu.SemaphoreType.DMA((NBUF,)),
            pltpu.SemaphoreType.DMA((NBUF,)),
        ],
    )
    return f(x, weights, node_order)


# speedup vs baseline: 2.6912x; 1.0051x over previous
import jax
import jax.numpy as jnp
from jax import lax
from jax.experimental import pallas as pl
from jax.experimental.pallas import tpu as pltpu
from jax.experimental.pallas import tpu_sc as plsc

L = 16
NC = 2
NS = 16
NW = NC * NS
CHUNK = 128
NBUF = 2
SIZE_OUT = 128


def _sc_body(x_hbm, w_hbm, no_hbm, out_hbm, w_v, no_v, xbufs, obufs,
             insem, outsem):
    size_in = x_hbm.shape[1]
    nvec_in = size_in // L
    rows_per_w = x_hbm.shape[0] // NW
    nch = rows_per_w // CHUNK

    wid = lax.axis_index("s") * NC + lax.axis_index("c")
    base = wid * rows_per_w

    pltpu.sync_copy(w_hbm, w_v)
    pltpu.sync_copy(no_hbm, no_v)

    zeros = jnp.zeros((L,), jnp.float32)
    for b in range(NBUF):
        @plsc.parallel_loop(0, CHUNK, unroll=4)
        def _(r, b=b):
            for k in range(SIZE_OUT // L):
                obufs[b][r, pl.ds(k * L, L)] = zeros

    wvecs = [w_v[pl.ds(j * L, L)] for j in range(nvec_in)]
    cvecs = [no_v[pl.ds(j * L, L)] for j in range(nvec_in)]

    in_cp = [
        pltpu.make_async_copy(
            x_hbm.at[pl.ds(base + ch * CHUNK, CHUNK)], xbufs[ch % NBUF],
            insem.at[ch % NBUF])
        for ch in range(nch)
    ]
    out_cp = [
        pltpu.make_async_copy(
            obufs[ch % NBUF], out_hbm.at[pl.ds(base + ch * CHUNK, CHUNK)],
            outsem.at[ch % NBUF])
        for ch in range(nch)
    ]

    in_cp[0].start()
    for ch in range(nch):
        b = ch % NBUF
        if ch + 1 < nch:
            in_cp[ch + 1].start()
        in_cp[ch].wait()
        if ch >= NBUF:
            out_cp[ch - NBUF].wait()

        @plsc.parallel_loop(0, CHUNK, unroll=4)
        def _(r, b=b):
            ridx = jnp.full((L,), r, jnp.int32)
            for j in range(nvec_in):
                v = xbufs[b][r, pl.ds(j * L, L)] * wvecs[j]
                plsc.store_scatter(obufs[b], [ridx, cvecs[j]], v)

        out_cp[ch].start()
    for ch in range(max(0, nch - NBUF), nch):
        out_cp[ch].wait()


def _body(x_hbm, w_hbm, no_hbm, out_hbm, w_v, no_v, xbuf0, xbuf1,
          obuf0, obuf1, insem, outsem):
    _sc_body(x_hbm, w_hbm, no_hbm, out_hbm, w_v, no_v,
             (xbuf0, xbuf1), (obuf0, obuf1), insem, outsem)


@jax.jit
def kernel(x, weights, node_order):
    B = x.shape[0]
    mesh = plsc.VectorSubcoreMesh(core_axis_name="c", subcore_axis_name="s")
    f = pl.kernel(
        _body,
        out_type=jax.ShapeDtypeStruct((B, SIZE_OUT), x.dtype),
        mesh=mesh,
        compiler_params=pltpu.CompilerParams(needs_layout_passes=False, skip_device_barrier=True),
        scratch_types=[
            pltpu.VMEM((x.shape[1],), jnp.float32),
            pltpu.VMEM((x.shape[1],), jnp.int32),
            pltpu.VMEM((CHUNK, x.shape[1]), jnp.float32),
            pltpu.VMEM((CHUNK, x.shape[1]), jnp.float32),
            pltpu.VMEM((CHUNK, SIZE_OUT), jnp.float32),
            pltpu.VMEM((CHUNK, SIZE_OUT), jnp.float32),
            pltpu.SemaphoreType.DMA((NBUF,)),
            pltpu.SemaphoreType.DMA((NBUF,)),
        ],
    )
    return f(x, weights, node_order)
